# baseline (speedup 1.0000x reference)
import functools

import jax
import jax.numpy as jnp
from jax import lax
from jax.experimental import pallas as pl
from jax.experimental.pallas import tpu as pltpu
from jax.experimental.pallas import tpu_sc as plsc

_NC = 2
_NS = 16
_L = 16
_NW = _NC * _NS
_CH = 4


@functools.lru_cache(maxsize=None)
def _build(B, T, V, D):
    N = B * T
    t_per_w = T // _NW
    CH = _CH
    NS = 8
    PD = 2
    n_chunks = t_per_w // CH
    assert t_per_w * _NW == T and n_chunks * CH == t_per_w
    assert D % _L == 0 and t_per_w % 8 == 0
    assert n_chunks % NS == 0 and n_chunks >= NS

    mesh = plsc.VectorSubcoreMesh(core_axis_name="c", subcore_axis_name="s")

    @functools.partial(
        pl.kernel,
        out_type=jax.ShapeDtypeStruct((N, D), jnp.float32),
        mesh=mesh,
        scratch_types=[
            pltpu.VMEM((B * t_per_w,), jnp.int32),
            pltpu.VMEM((NS, B * CH, D), jnp.float32),
            pltpu.VMEM((NS, CH, D), jnp.float32),
            pltpu.SemaphoreType.DMA((NS,)),
            pltpu.SemaphoreType.DMA((NS,)),
            pltpu.SemaphoreType.DMA((NS,)),
        ],
    )
    def emb_kernel(idx_hbm, tok_hbm, pos_hbm, out_hbm,
                   idx_v, tokb, posb, gsem, psem, osem):
        wid = lax.axis_index("s") * _NC + lax.axis_index("c")
        t_base = wid * t_per_w

        pltpu.sync_copy(idx_hbm.at[pl.ds(wid * (B * t_per_w), B * t_per_w)],
                        idx_v)

        def gather_desc(cc, s):
            return pltpu.make_async_copy(
                tok_hbm.at[idx_v.at[pl.ds(cc * (B * CH), B * CH)]],
                tokb.at[s], gsem.at[s])

        def pos_desc(cc, s):
            return pltpu.make_async_copy(
                pos_hbm.at[pl.ds(t_base + cc * CH, CH)], posb.at[s], psem.at[s])

        def out_desc(cc, s, b):
            return pltpu.make_async_copy(
                tokb.at[s, pl.ds(b * CH, CH)],
                out_hbm.at[pl.ds(b * T + t_base + cc * CH, CH)],
                osem.at[s])

        for s in range(PD):
            pos_desc(s, s).start()
            gather_desc(s, s).start()

        @pl.loop(0, n_chunks, step=NS)
        def _(c):
            for s in range(NS):
                cc = c + s
                pos_desc(cc, s).wait()
                gather_desc(cc, s).wait()

                @pl.loop(0, CH)
                def _(r):
                    for j in range(D // _L):
                        sl = pl.ds(j * _L, _L)
                        p = posb[s, r, sl]
                        for b in range(B):
                            tokb[s, b * CH + r, sl] = tokb[s, b * CH + r, sl] + p

                for b in range(B):
                    out_desc(cc, s, b).start()

                sp = (s + PD) % NS

                @pl.when(cc + PD < n_chunks)
                def _():
                    @pl.when(cc >= NS - PD)
                    def _():
                        for b in range(B):
                            out_desc(cc - (NS - PD), sp, b).wait()

                    pos_desc(cc + PD, sp).start()
                    gather_desc(cc + PD, sp).start()

        for k in range(n_chunks - NS, n_chunks):
            for b in range(B):
                out_desc(k, k % NS, b).wait()

    return emb_kernel


def kernel(x, token_emb, pos_emb):
    B, T = x.shape
    V, D = token_emb.shape
    t_per_w = T // _NW
    xp = (x.astype(jnp.int32)
           .reshape(B, _NW, t_per_w // _CH, _CH)
           .transpose(1, 2, 0, 3)
           .reshape(-1))
    out = _build(B, T, V, D)(xp, token_emb, pos_emb)
    return out.reshape(B, T, D)


# speedup vs baseline: 1.3571x; 1.3571x over previous
import functools

import jax
import jax.numpy as jnp
from jax import lax
from jax.experimental import pallas as pl
from jax.experimental.pallas import tpu as pltpu
from jax.experimental.pallas import tpu_sc as plsc

_NC = 2
_NS = 16
_L = 16
_NW = _NC * _NS
_CH = 8


@functools.lru_cache(maxsize=None)
def _build(B, T, V, D):
    N = B * T
    t_per_w = T // _NW
    CH = _CH
    NS = 4
    PD = 3
    n_chunks = t_per_w // CH
    assert t_per_w * _NW == T and n_chunks * CH == t_per_w
    assert D % _L == 0 and t_per_w % 8 == 0
    assert n_chunks % NS == 0 and n_chunks >= NS

    mesh = plsc.VectorSubcoreMesh(core_axis_name="c", subcore_axis_name="s")

    @functools.partial(
        pl.kernel,
        out_type=jax.ShapeDtypeStruct((N, D), jnp.float32),
        mesh=mesh,
        scratch_types=[
            pltpu.VMEM((B * t_per_w,), jnp.int32),
            pltpu.VMEM((NS, B * CH, D), jnp.float32),
            pltpu.VMEM((NS, CH, D), jnp.float32),
            pltpu.SemaphoreType.DMA((NS,)),
            pltpu.SemaphoreType.DMA((NS,)),
            pltpu.SemaphoreType.DMA((NS,)),
        ],
    )
    def emb_kernel(idx_hbm, tok_hbm, pos_hbm, out_hbm,
                   idx_v, tokb, posb, gsem, psem, osem):
        wid = lax.axis_index("s") * _NC + lax.axis_index("c")
        t_base = wid * t_per_w

        pltpu.sync_copy(idx_hbm.at[pl.ds(wid * (B * t_per_w), B * t_per_w)],
                        idx_v)

        def gather_desc(cc, s):
            return pltpu.make_async_copy(
                tok_hbm.at[idx_v.at[pl.ds(cc * (B * CH), B * CH)]],
                tokb.at[s], gsem.at[s])

        def pos_desc(cc, s):
            return pltpu.make_async_copy(
                pos_hbm.at[pl.ds(t_base + cc * CH, CH)], posb.at[s], psem.at[s])

        def out_desc(cc, s, b):
            return pltpu.make_async_copy(
                tokb.at[s, pl.ds(b * CH, CH)],
                out_hbm.at[pl.ds(b * T + t_base + cc * CH, CH)],
                osem.at[s])

        for s in range(PD):
            pos_desc(s, s).start()
            gather_desc(s, s).start()

        @pl.loop(0, n_chunks, step=NS)
        def _(c):
            for s in range(NS):
                cc = c + s
                pos_desc(cc, s).wait()
                gather_desc(cc, s).wait()

                @pl.loop(0, CH)
                def _(r):
                    for j in range(D // _L):
                        sl = pl.ds(j * _L, _L)
                        p = posb[s, r, sl]
                        for b in range(B):
                            tokb[s, b * CH + r, sl] = tokb[s, b * CH + r, sl] + p

                for b in range(B):
                    out_desc(cc, s, b).start()

                sp = (s + PD) % NS

                @pl.when(cc + PD < n_chunks)
                def _():
                    @pl.when(cc >= NS - PD)
                    def _():
                        for b in range(B):
                            out_desc(cc - (NS - PD), sp, b).wait()

                    pos_desc(cc + PD, sp).start()
                    gather_desc(cc + PD, sp).start()

        for k in range(n_chunks - NS, n_chunks):
            for b in range(B):
                out_desc(k, k % NS, b).wait()

    return emb_kernel


def kernel(x, token_emb, pos_emb):
    B, T = x.shape
    V, D = token_emb.shape
    t_per_w = T // _NW
    xp = (x.astype(jnp.int32)
           .reshape(B, _NW, t_per_w // _CH, _CH)
           .transpose(1, 2, 0, 3)
           .reshape(-1))
    out = _build(B, T, V, D)(xp, token_emb, pos_emb)
    return out.reshape(B, T, D)
